# baseline (speedup 1.0000x reference)
import functools

import jax
import jax.numpy as jnp
from jax import lax
from jax.experimental import pallas as pl
from jax.experimental.pallas import tpu as pltpu
from jax.experimental.pallas import tpu_sc as plsc

_B, _C, _N = 16, 512, 2048
_MAX_H, _MAX_W = 70, 40
_GRID = 300
_HW = _MAX_H * _MAX_W
_HWP = 2816
_NT = _N + 8
_NC, _NS = 2, 16
_PPT = _HW // _NC
_P = 56
_NCHUNK = _PPT // _P


def _tc_body(feat_ref, ys_ref, xs_ref, ft_ref, pix_ref):
    yb = ys_ref[0]
    xb = xs_ref[0]
    valid = yb > -1
    min_y = jnp.min(jnp.where(valid, yb, _GRID))
    max_y = jnp.max(jnp.where(valid, yb, -1))
    min_x = jnp.min(jnp.where(valid, xb, _GRID))
    max_x = jnp.max(jnp.where(valid, xb, -1))
    h0 = max_y - min_y + 1
    w0 = max_x - min_x + 1
    swap = w0 > h0
    height = jnp.where(swap, w0, h0)
    width = jnp.where(swap, h0, w0)
    h_dif = height - _MAX_H
    w_dif = width - _MAX_W
    cut_top = jnp.where(h_dif > 0, (h_dif + 1) // 2, 0)
    pad_top = jnp.where(h_dif > 0, 0, (-h_dif + 1) // 2)
    cut_left = jnp.where(w_dif > 0, (w_dif + 1) // 2, 0)
    pad_right = jnp.where(w_dif > 0, 0, (-w_dif + 1) // 2)
    ry = yb - min_y
    rx = xb - min_x
    row = jnp.where(swap, rx, ry)
    col = jnp.where(swap, ry, rx)
    r = row - cut_top + pad_top
    c = col - cut_left + pad_right
    inb = (r >= 0) & (r < _MAX_H) & (c >= 0) & (c < _MAX_W)
    f = feat_ref[0]
    real = jnp.min(jnp.where(f != -1.0, 1, 0).astype(jnp.int32),
                   axis=0, keepdims=True)
    pix = jnp.where(valid & inb & (real > 0), r * _MAX_W + c, -1)
    pix = pix.astype(jnp.int32)
    for t in range(_N // 128):
        pix_ref[0, pl.ds(t, 1), :] = pix[:, t * 128:(t + 1) * 128]
    ft_ref[0, pl.ds(0, _N), :] = jnp.transpose(f)
    ft_ref[0, pl.ds(_N, 8), :] = jnp.zeros((8, _C), jnp.float32)


def _tc_pass(features, ys3, xs3):
    return pl.pallas_call(
        _tc_body,
        grid=(_B,),
        in_specs=[
            pl.BlockSpec((1, _C, _N), lambda b: (b, 0, 0)),
            pl.BlockSpec((1, 1, _N), lambda b: (b, 0, 0)),
            pl.BlockSpec((1, 1, _N), lambda b: (b, 0, 0)),
        ],
        out_specs=[
            pl.BlockSpec((1, _NT, _C), lambda b: (b, 0, 0)),
            pl.BlockSpec((1, _N // 128, 128), lambda b: (b, 0, 0)),
        ],
        out_shape=[
            jax.ShapeDtypeStruct((_B, _NT, _C), jnp.float32),
            jax.ShapeDtypeStruct((_B, _N // 128, 128), jnp.int32),
        ],
    )(features, ys3, xs3)


@functools.lru_cache(maxsize=None)
def _sc_gather_fn():
    return functools.partial(
        pl.kernel,
        mesh=plsc.VectorSubcoreMesh(core_axis_name="c", subcore_axis_name="s"),
        compiler_params=pltpu.CompilerParams(needs_layout_passes=False),
        out_type=jax.ShapeDtypeStruct((_B, _HW, _C), jnp.float32),
        scratch_types=[
            pltpu.VMEM((_N // 128, 128), jnp.int32),
            pltpu.VMEM((_HWP,), jnp.int32),
            pltpu.VMEM((_P, _C), jnp.float32),
            pltpu.VMEM((_P, _C), jnp.float32),
            pltpu.SemaphoreType.DMA,
            pltpu.SemaphoreType.DMA,
            pltpu.SemaphoreType.DMA,
            pltpu.SemaphoreType.DMA,
        ],
    )(_sc_gather_body)


def _sc_gather_body(ft_hbm, pix_hbm, out_hbm, pix_v, imap_v,
                    rows0, rows1, sin0, sin1, sout0, sout1):
    cid = lax.axis_index("c")
    sid = lax.axis_index("s")
    b = sid
    p0c = cid * _PPT
    rows = (rows0, rows1)
    sins, souts = (sin0, sin1), (sout0, sout1)

    pltpu.sync_copy(pix_hbm.at[b], pix_v)
    zvec = jnp.full((16,), _N, jnp.int32)
    for k in range(_HWP // 16):
        imap_v[pl.ds(k * 16, 16)] = zvec
    iota16 = lax.iota(jnp.int32, 16)
    for j in range(_N // 16):
        idx = pix_v[j // 8, pl.ds((j % 8) * 16, 16)]
        m = idx >= 0
        v = iota16 + (j * 16)
        plsc.store_scatter(imap_v, [jnp.maximum(idx, 0)], v, mask=m)

    def start_in(i, par):
        pltpu.make_async_copy(
            ft_hbm.at[b].at[imap_v.at[pl.ds(p0c + i * _P, _P)]],
            rows[par], sins[par]).start()

    def wait_in(i, par):
        pltpu.make_async_copy(
            ft_hbm.at[b].at[imap_v.at[pl.ds(p0c + i * _P, _P)]],
            rows[par], sins[par]).wait()

    def start_out(i, par):
        pltpu.make_async_copy(
            rows[par], out_hbm.at[b, pl.ds(p0c + i * _P, _P)],
            souts[par]).start()

    def wait_out(i, par):
        pltpu.make_async_copy(
            rows[par], out_hbm.at[b, pl.ds(p0c + i * _P, _P)],
            souts[par]).wait()

    start_in(0, 0)
    start_in(1, 1)

    def body2(j, carry):
        for par in (0, 1):
            i = 2 * j + par
            wait_in(i, par)

            @pl.when(i >= 2)
            def _():
                wait_out(i - 2, par)

            start_out(i, par)

            @pl.when(i + 2 < _NCHUNK)
            def _():
                start_in(i + 2, par)
        return carry

    lax.fori_loop(0, _NCHUNK // 2, body2, 0)
    i_last = _NCHUNK - 1
    wait_in(i_last, 0)
    wait_out(i_last - 2, 0)
    start_out(i_last, 0)
    wait_out(i_last - 1, 1)
    wait_out(i_last, 0)


def kernel(features, ys, xs):
    ys3 = ys.reshape(_B, 1, _N)
    xs3 = xs.reshape(_B, 1, _N)
    ft, pix = _tc_pass(features, ys3, xs3)
    out = _sc_gather_fn()(ft, pix)
    return out.reshape(_B, _MAX_H, _MAX_W, _C).transpose(0, 3, 1, 2)


# speedup vs baseline: 3.9522x; 3.9522x over previous
import functools

import jax
import jax.numpy as jnp
from jax import lax
from jax.experimental import pallas as pl
from jax.experimental.pallas import tpu as pltpu
from jax.experimental.pallas import tpu_sc as plsc

_B, _C, _N = 16, 512, 2048
_MAX_H, _MAX_W = 70, 40
_GRID = 300
_HW = _MAX_H * _MAX_W
_HWP = 2816
_NT = _N + 8
_NC, _NS = 2, 16
_PPT = _HW // _NC
_P = 56
_NCHUNK = _PPT // _P


def _tc_body(feat_ref, ys_ref, xs_ref, ft_ref, pix_ref):
    yb = ys_ref[0]
    xb = xs_ref[0]
    valid = yb > -1
    min_y = jnp.min(jnp.where(valid, yb, _GRID))
    max_y = jnp.max(jnp.where(valid, yb, -1))
    min_x = jnp.min(jnp.where(valid, xb, _GRID))
    max_x = jnp.max(jnp.where(valid, xb, -1))
    h0 = max_y - min_y + 1
    w0 = max_x - min_x + 1
    swap = w0 > h0
    height = jnp.where(swap, w0, h0)
    width = jnp.where(swap, h0, w0)
    h_dif = height - _MAX_H
    w_dif = width - _MAX_W
    cut_top = jnp.where(h_dif > 0, (h_dif + 1) // 2, 0)
    pad_top = jnp.where(h_dif > 0, 0, (-h_dif + 1) // 2)
    cut_left = jnp.where(w_dif > 0, (w_dif + 1) // 2, 0)
    pad_right = jnp.where(w_dif > 0, 0, (-w_dif + 1) // 2)
    ry = yb - min_y
    rx = xb - min_x
    row = jnp.where(swap, rx, ry)
    col = jnp.where(swap, ry, rx)
    r = row - cut_top + pad_top
    c = col - cut_left + pad_right
    inb = (r >= 0) & (r < _MAX_H) & (c >= 0) & (c < _MAX_W)
    f = feat_ref[0]
    real = jnp.min(jnp.where(f != -1.0, 1, 0).astype(jnp.int32),
                   axis=0, keepdims=True)
    pix = jnp.where(valid & inb & (real > 0), r * _MAX_W + c, -1)
    pix = pix.astype(jnp.int32)
    for t in range(_N // 128):
        pix_ref[0, pl.ds(t, 1), :] = pix[:, t * 128:(t + 1) * 128]
    ft_ref[0, pl.ds(0, _N), :] = jnp.transpose(f)
    ft_ref[0, pl.ds(_N, 8), :] = jnp.zeros((8, _C), jnp.float32)


def _tc_pass(features, ys3, xs3):
    return pl.pallas_call(
        _tc_body,
        grid=(_B,),
        in_specs=[
            pl.BlockSpec((1, _C, _N), lambda b: (b, 0, 0)),
            pl.BlockSpec((1, 1, _N), lambda b: (b, 0, 0)),
            pl.BlockSpec((1, 1, _N), lambda b: (b, 0, 0)),
        ],
        out_specs=[
            pl.BlockSpec((1, _NT, _C), lambda b: (b, 0, 0)),
            pl.BlockSpec((1, _N // 128, 128), lambda b: (b, 0, 0)),
        ],
        out_shape=[
            jax.ShapeDtypeStruct((_B, _NT, _C), jnp.float32),
            jax.ShapeDtypeStruct((_B, _N // 128, 128), jnp.int32),
        ],
    )(features, ys3, xs3)


_KMAX = _N
_PCH = 64
_NROW = _KMAX // _PCH


@functools.lru_cache(maxsize=None)
def _sc_gather_fn():
    return functools.partial(
        pl.kernel,
        mesh=plsc.VectorSubcoreMesh(core_axis_name="c", subcore_axis_name="s"),
        compiler_params=pltpu.CompilerParams(needs_layout_passes=False),
        out_type=jax.ShapeDtypeStruct((_B, _HW, _C), jnp.float32),
        scratch_types=[
            pltpu.VMEM((_N // 128, 128), jnp.int32),
            pltpu.VMEM((_KMAX + 16,), jnp.int32),
            pltpu.VMEM((_KMAX + 16,), jnp.int32),
            pltpu.VMEM((_NROW, _PCH), jnp.int32),
            pltpu.VMEM((_NROW, _PCH), jnp.int32),
            pltpu.VMEM((_P, _C), jnp.float32),
            pltpu.VMEM((_PCH, _C), jnp.float32),
            pltpu.SemaphoreType.DMA,
            pltpu.SemaphoreType.DMA,
            pltpu.SemaphoreType.DMA,
        ],
    )(_sc_gather_body)


def _sc_gather_body(ft_hbm, pix_hbm, out_hbm, pix_v, plist_f, ilist_f,
                    plist2, ilist2, zbuf, rows, szero, sin, sout):
    cid = lax.axis_index("c")
    sid = lax.axis_index("s")
    b = sid
    p0c = cid * _PPT

    z16 = jnp.zeros((16,), jnp.float32)
    for r in range(_P):
        for q in range(_C // 16):
            zbuf[r, pl.ds(q * 16, 16)] = z16
    for k in range(_NCHUNK):
        pltpu.make_async_copy(
            zbuf, out_hbm.at[b, pl.ds(p0c + k * _P, _P)], szero).start()

    pltpu.sync_copy(pix_hbm.at[b], pix_v)
    iota16 = lax.iota(jnp.int32, 16)
    off = jnp.int32(0)
    for j in range(_N // 16):
        idx = pix_v[j // 8, pl.ds((j % 8) * 16, 16)]
        m = (idx >= p0c) & (idx < p0c + _PPT)
        plsc.store_compressed(plist_f.at[pl.ds(off, 16)], idx, mask=m)
        plsc.store_compressed(ilist_f.at[pl.ds(off, 16)], iota16 + (j * 16), mask=m)
        off = off + plsc.all_reduce_population_count(m)[0]
    kcnt = off

    p16 = plist_f[pl.ds(0, 16)]
    i16 = ilist_f[pl.ds(0, 16)]
    pad_p = jnp.full((16,), 0, jnp.int32) + p16[0]
    pad_i = jnp.full((16,), 0, jnp.int32) + i16[0]
    for r in range(_NROW):
        for q in range(_PCH // 16):
            flat = r * _PCH + q * 16
            pv = plist_f[pl.ds(flat, 16)]
            iv = ilist_f[pl.ds(flat, 16)]
            valid = (iota16 + flat) < kcnt
            plist2[r, pl.ds(q * 16, 16)] = jnp.where(valid, pv, pad_p)
            ilist2[r, pl.ds(q * 16, 16)] = jnp.where(valid, iv, pad_i)

    for k in range(_NCHUNK):
        pltpu.make_async_copy(
            zbuf, out_hbm.at[b, pl.ds(p0c + k * _P, _P)], szero).wait()

    nch = (kcnt + (_PCH - 1)) >> 6

    def chunk_body(c, carry):
        pltpu.async_copy(ft_hbm.at[b].at[ilist2.at[c]], rows, sin).wait()
        pltpu.async_copy(rows, out_hbm.at[b].at[plist2.at[c]], sout).wait()
        return carry

    lax.fori_loop(0, nch, chunk_body, 0)


def kernel(features, ys, xs):
    ys3 = ys.reshape(_B, 1, _N)
    xs3 = xs.reshape(_B, 1, _N)
    ft, pix = _tc_pass(features, ys3, xs3)
    out = _sc_gather_fn()(ft, pix)
    return out.reshape(_B, _MAX_H, _MAX_W, _C).transpose(0, 3, 1, 2)
